# baseline (speedup 1.0000x reference)
import jax
import jax.numpy as jnp
from jax import lax
from jax.experimental import pallas as pl
from jax.experimental.pallas import tpu as pltpu
from jax.experimental.pallas import tpu_sc as plsc

VOCAB = 50000
PROMPT_LEN = 100
D_MODEL = 1024
BATCH = 4
SEQ = 8192
NTOK = BATCH * SEQ

_info = plsc.get_sparse_core_info()
NC, NS, L = _info.num_cores, _info.num_subcores, _info.num_lanes
NW = NC * NS
TPW = NTOK // NW
NGRP = TPW // L
R = 16
NSUB = TPW // R
NBUF = 4


def _body(ids_hbm, embed_hbm, prompt_hbm, out_hbm,
          raw_v, safe_v, pos_v, pid_v, bufs_v, pbuf_v, sem1, *semgw):
  semg = semgw[:NBUF]
  semw = semgw[NBUF:]
  wid = lax.axis_index("s") * NC + lax.axis_index("c")
  base = wid * TPW

  pltpu.sync_copy(ids_hbm.at[pl.ds(base, TPW)], raw_v)

  def safe_grp(g):
    v = raw_v[pl.ds(g * L, L)]
    safe_v[pl.ds(g * L, L)] = jnp.where(v >= VOCAB, v - VOCAB, v)

  def compact_grp(g, k):
    v = raw_v[pl.ds(g * L, L)]
    mask = v >= VOCAB
    mi = jnp.where(mask, 1, 0).astype(jnp.int32)
    tgt = k + plsc.cumsum(mi) - 1
    pos = base + g * L + lax.iota(jnp.int32, L)
    pid = jnp.clip(v - VOCAB, 0, PROMPT_LEN - 1)
    plsc.store_scatter(pos_v, [tgt], pos, mask=mask)
    plsc.store_scatter(pid_v, [tgt], pid, mask=mask)
    return k + jnp.sum(mi)

  def gsrc(s):
    return embed_hbm.at[safe_v.at[pl.ds(s * R, R)]]

  def wdst(s):
    return out_hbm.at[pl.ds(base + s * R, R)]

  for b in range(NBUF):
    safe_grp(b)
    pltpu.async_copy(gsrc(b), bufs_v.at[b], semg[b])

  NRND = NSUB // NBUF

  def rnd(j, k):
    for b in range(NBUF):
      s = j * NBUF + b
      k = compact_grp(s, k)

      @pl.when(j < NRND - 1)
      def _():
        safe_grp(s + NBUF)

      pltpu.make_async_copy(gsrc(s), bufs_v.at[b], semg[b]).wait()
      pltpu.async_copy(bufs_v.at[b], wdst(s), semw[b])

      @pl.when(j < NRND - 1)
      def _():
        pltpu.make_async_copy(bufs_v.at[b], wdst(s), semw[b]).wait()
        pltpu.async_copy(gsrc(s + NBUF), bufs_v.at[b], semg[b])

    return k

  k = lax.fori_loop(0, NRND, rnd, jnp.int32(0))
  for b in range(NBUF):
    pltpu.make_async_copy(bufs_v.at[b], wdst(NSUB - NBUF + b), semw[b]).wait()

  @pl.when(k > 0)
  def _():
    lane = lax.iota(jnp.int32, L)
    lane0 = lane == 0
    e0pos = jnp.sum(jnp.where(lane0, pos_v[pl.ds(0, L)], 0))
    e0pid = jnp.sum(jnp.where(lane0, pid_v[pl.ds(0, L)], 0))
    fill_idx = k + lane
    plsc.store_scatter(pos_v, [fill_idx], jnp.zeros((L,), jnp.int32) + e0pos)
    plsc.store_scatter(pid_v, [fill_idx], jnp.zeros((L,), jnp.int32) + e0pid)

    nch = (k + L - 1) // L

    def fix(j, _):
      pidx = pid_v[pl.ds(j * L, L)]
      posx = pos_v[pl.ds(j * L, L)]
      pltpu.async_copy(prompt_hbm.at[pidx], pbuf_v, sem1).wait()
      pltpu.async_copy(pbuf_v, out_hbm.at[posx], sem1).wait()
      return 0

    lax.fori_loop(0, nch, fix, 0)


@jax.jit
def _run(ids_flat, embed_table, prompt_table):
  mesh = plsc.VectorSubcoreMesh(core_axis_name="c", subcore_axis_name="s")
  f = pl.kernel(
      _body,
      out_type=jax.ShapeDtypeStruct((NTOK, D_MODEL), jnp.float32),
      mesh=mesh,
      compiler_params=pltpu.CompilerParams(
          needs_layout_passes=False, skip_device_barrier=True),
      scratch_types=[
          pltpu.VMEM((TPW,), jnp.int32),
          pltpu.VMEM((TPW,), jnp.int32),
          pltpu.VMEM((TPW + L,), jnp.int32),
          pltpu.VMEM((TPW + L,), jnp.int32),
          pltpu.VMEM((NBUF, R, D_MODEL), jnp.float32),
          pltpu.VMEM((L, D_MODEL), jnp.float32),
          pltpu.SemaphoreType.DMA,
      ] + [pltpu.SemaphoreType.DMA] * (2 * NBUF),
  )
  return f(ids_flat, embed_table, prompt_table)


def kernel(input_ids, labels, embed_table, prompt_table):
  del labels
  out = _run(input_ids.reshape(-1), embed_table, prompt_table)
  return out.reshape(BATCH, SEQ, D_MODEL)


# speedup vs baseline: 1.0022x; 1.0022x over previous
import jax
import jax.numpy as jnp
from jax import lax
from jax.experimental import pallas as pl
from jax.experimental.pallas import tpu as pltpu
from jax.experimental.pallas import tpu_sc as plsc

VOCAB = 50000
PROMPT_LEN = 100
D_MODEL = 1024
BATCH = 4
SEQ = 8192
NTOK = BATCH * SEQ

_info = plsc.get_sparse_core_info()
NC, NS, L = _info.num_cores, _info.num_subcores, _info.num_lanes
NW = NC * NS
TPW = NTOK // NW
NGRP = TPW // L
R = 16
NSUB = TPW // R
NBUF = 4


def _body(ids_hbm, embed_hbm, prompt_hbm, out_hbm,
          raw_v, safe_v, pos_v, pid_v, bufs_v, pbuf_v, sem1, *semgw):
  semg = semgw[:NBUF]
  semw = semgw[NBUF:]
  wid = lax.axis_index("s") * NC + lax.axis_index("c")
  base = wid * TPW

  pltpu.sync_copy(ids_hbm.at[pl.ds(base, TPW)], raw_v)

  def safe_grp(g):
    v = raw_v[pl.ds(g * L, L)]
    safe_v[pl.ds(g * L, L)] = jnp.where(v >= VOCAB, v - VOCAB, v)

  def compact_grp(g, k):
    v = raw_v[pl.ds(g * L, L)]
    mask = v >= VOCAB
    mi = jnp.where(mask, 1, 0).astype(jnp.int32)
    tgt = k + plsc.cumsum(mi) - 1
    pos = base + g * L + lax.iota(jnp.int32, L)
    pid = jnp.clip(v - VOCAB, 0, PROMPT_LEN - 1)
    plsc.store_scatter(pos_v, [tgt], pos, mask=mask)
    plsc.store_scatter(pid_v, [tgt], pid, mask=mask)
    return k + jnp.sum(mi)

  def gsrc(s):
    return embed_hbm.at[safe_v.at[pl.ds(s * R, R)]]

  def wdst(s):
    return out_hbm.at[pl.ds(base + s * R, R)]

  for b in range(NBUF):
    safe_grp(b)
    pltpu.async_copy(gsrc(b), bufs_v.at[b], semg[b])

  NRND = NSUB // NBUF

  def rnd(j, k):
    for b in range(NBUF):
      s = j * NBUF + b
      k = compact_grp(s, k)

      @pl.when(j < NRND - 1)
      def _():
        safe_grp(s + NBUF)

      pltpu.make_async_copy(gsrc(s), bufs_v.at[b], semg[b]).wait()
      pltpu.async_copy(bufs_v.at[b], wdst(s), semw[b])

      @pl.when(j < NRND - 1)
      def _():
        pltpu.make_async_copy(bufs_v.at[b], wdst(s), semw[b]).wait()
        pltpu.async_copy(gsrc(s + NBUF), bufs_v.at[b], semg[b])

    return k

  k = lax.fori_loop(0, NRND, rnd, jnp.int32(0))
  for b in range(NBUF):
    pltpu.make_async_copy(bufs_v.at[b], wdst(NSUB - NBUF + b), semw[b]).wait()

  @pl.when(k > 0)
  def _():
    lane = lax.iota(jnp.int32, L)
    lane0 = lane == 0
    e0pos = jnp.sum(jnp.where(lane0, pos_v[pl.ds(0, L)], 0))
    e0pid = jnp.sum(jnp.where(lane0, pid_v[pl.ds(0, L)], 0))
    fill_idx = k + lane
    plsc.store_scatter(pos_v, [fill_idx], jnp.zeros((L,), jnp.int32) + e0pos)
    plsc.store_scatter(pid_v, [fill_idx], jnp.zeros((L,), jnp.int32) + e0pid)

    nch = (k + L - 1) // L

    def fix(j, _):
      pidx = pid_v[pl.ds(j * L, L)]
      posx = pos_v[pl.ds(j * L, L)]
      pltpu.async_copy(prompt_hbm.at[pidx], pbuf_v, sem1).wait()
      pltpu.async_copy(pbuf_v, out_hbm.at[posx], sem1).wait()
      return 0

    lax.fori_loop(0, nch, fix, 0)


@jax.jit
def _run(ids_flat, embed_table, prompt_table):
  mesh = plsc.VectorSubcoreMesh(core_axis_name="c", subcore_axis_name="s")
  f = pl.kernel(
      _body,
      out_type=jax.ShapeDtypeStruct((NTOK, D_MODEL), jnp.float32),
      mesh=mesh,
      compiler_params=pltpu.CompilerParams(needs_layout_passes=False),
      scratch_types=[
          pltpu.VMEM((TPW,), jnp.int32),
          pltpu.VMEM((TPW,), jnp.int32),
          pltpu.VMEM((TPW + L,), jnp.int32),
          pltpu.VMEM((TPW + L,), jnp.int32),
          pltpu.VMEM((NBUF, R, D_MODEL), jnp.float32),
          pltpu.VMEM((L, D_MODEL), jnp.float32),
          pltpu.SemaphoreType.DMA,
      ] + [pltpu.SemaphoreType.DMA] * (2 * NBUF),
  )
  return f(ids_flat, embed_table, prompt_table)


def kernel(input_ids, labels, embed_table, prompt_table):
  del labels
  out = _run(input_ids.reshape(-1), embed_table, prompt_table)
  return out.reshape(BATCH, SEQ, D_MODEL)
